# baseline (speedup 1.0000x reference)
import functools

import jax
import jax.numpy as jnp
import numpy as np
from jax import lax
from jax.experimental import pallas as pl
from jax.experimental.pallas import tpu as pltpu
from jax.experimental.pallas import tpu_sc as plsc

NC = 2
NS = 16
NW = NC * NS
LANES = 16


def _prep_idx(inputsT, QS):
    L, B = inputsT.shape
    PAD = 64
    BK = 2048
    H = B // 2

    def remap(x):
        q = ((x >= QS).astype(jnp.int32) + (x >= 2 * QS).astype(jnp.int32)
             + (x >= 3 * QS).astype(jnp.int32))
        return 4 * (x - q * QS) + q

    def body(xa_ref, xb_ref, o_ref):
        ya = remap(jnp.transpose(xa_ref[...]))
        yb = remap(jnp.transpose(xb_ref[...]))
        z = jnp.zeros((BK, PAD - L), jnp.int32)
        o_ref[...] = jnp.concatenate([ya, z, yb, z], axis=1)

    return pl.pallas_call(
        body,
        grid=(H // BK,),
        in_specs=[
            pl.BlockSpec((L, BK), lambda i: (0, i)),
            pl.BlockSpec((L, BK), lambda i, o=H // BK: (0, i + o)),
        ],
        out_specs=pl.BlockSpec((BK, 2 * PAD), lambda i: (i, 0)),
        out_shape=jax.ShapeDtypeStruct((H, 2 * PAD), jnp.int32),
    )(inputsT, inputsT)


@functools.partial(jax.jit, static_argnums=(2, 3))
def _sc_embed_sum(idx2d, table, B, L):
    W = table.shape[1]
    EMB = 2 * W
    PAD = 64
    EPW = B // NW
    CH_E = 8
    CH_R = CH_E * PAD
    NCH = EPW // CH_E
    HW = NW // 2

    mesh = plsc.VectorSubcoreMesh(core_axis_name="c", subcore_axis_name="s")

    @functools.partial(
        pl.kernel,
        out_type=jax.ShapeDtypeStruct((B, EMB), jnp.float32),
        mesh=mesh,
        scratch_types=[
            pltpu.VMEM((4, CH_E, PAD), jnp.int32),
            pltpu.VMEM((4, CH_R, W), jnp.uint32),
            pltpu.VMEM((4, CH_E, EMB), jnp.float32),
        ] + [pltpu.SemaphoreType.DMA] * 12,
        compiler_params=pltpu.CompilerParams(
            use_tc_tiling_on_sc=False, needs_layout_passes=False),
    )
    def sc_kernel(idx_hbm, tab_hbm, out_hbm, idxs, rows, outs, *sems):
        wid = lax.axis_index("s") * NC + lax.axis_index("c")
        half = wid // HW
        hw = wid - half * HW
        row_base = hw * EPW
        lane0 = half * PAD
        out_base = half * (B // 2) + hw * EPW
        si = sems[0:4]
        sg = sems[4:8]
        so = sems[8:12]
        himask = jnp.uint32(0xFFFF0000)

        def issue_idx(c, s):
            pltpu.async_copy(
                idx_hbm.at[pl.ds(row_base + c * CH_E, CH_E),
                           pl.ds(lane0, PAD)],
                idxs.at[s], si[s])

        def wait_idx(s):
            pltpu.make_async_copy(
                idx_hbm.at[pl.ds(0, CH_E), pl.ds(0, PAD)],
                idxs.at[s], si[s]).wait()

        def issue_gathers(s):
            for e in range(CH_E):
                pltpu.async_copy(
                    tab_hbm.at[idxs.at[s].at[e]],
                    rows.at[s].at[pl.ds(e * PAD, PAD)],
                    sg[s])

        def wait_gathers(s):
            pltpu.make_async_copy(
                tab_hbm.at[pl.ds(0, CH_R)], rows.at[s], sg[s]).wait()

        def issue_out(c, s):
            pltpu.async_copy(
                outs.at[s], out_hbm.at[pl.ds(out_base + c * CH_E, CH_E)],
                so[s])

        def wait_out(s):
            pltpu.make_async_copy(
                outs.at[s], out_hbm.at[pl.ds(0, CH_E)], so[s]).wait()

        def expand(w):
            lo = plsc.bitcast(w << 16, jnp.float32)
            hi = plsc.bitcast(w & himask, jnp.float32)
            return lo, hi

        def reduce_chunk(s):
            rows_s = rows.at[s]
            outs_s = outs.at[s]

            @pl.loop(0, CH_E)
            def _(e):
                r0 = e * PAD
                accs = [None] * (2 * W // LANES)
                accs2 = [None] * (2 * W // LANES)
                for j in range(L):
                    tgt = accs if (j & 1) == 0 else accs2
                    for h in range(W // LANES):
                        lo, hi = expand(
                            rows_s[r0 + j, pl.ds(h * LANES, LANES)])
                        if tgt[2 * h] is None:
                            tgt[2 * h] = lo
                            tgt[2 * h + 1] = hi
                        else:
                            tgt[2 * h] = tgt[2 * h] + lo
                            tgt[2 * h + 1] = tgt[2 * h + 1] + hi
                for k in range(len(accs)):
                    outs_s[e, pl.ds(k * LANES, LANES)] = accs[k] + accs2[k]

        for k in range(4):
            issue_idx(k, k)
        wait_idx(0)
        issue_gathers(0)
        wait_idx(1)
        issue_gathers(1)

        @pl.loop(0, NCH, step=4)
        def _(cbase):
            for b in range(4):
                c = cbase + b
                s = b
                wait_gathers(s)

                @pl.when(c + 4 < NCH)
                def _():
                    issue_idx(c + 4, s)

                @pl.when(c + 2 < NCH)
                def _():
                    wait_idx((b + 2) % 4)
                    issue_gathers((b + 2) % 4)

                @pl.when(c >= 4)
                def _():
                    wait_out(s)

                reduce_chunk(s)
                issue_out(c, s)

        for k in range(4):
            wait_out(k)

    return sc_kernel(idx2d, table)


_REPACK_BN = 15872
_NQ = 4


def _repack_split(V):
    nb = -(-V // (_NQ * _REPACK_BN))
    return nb * _REPACK_BN


def _repack_table(tableT):
    EMB, V = tableT.shape
    BN = _REPACK_BN
    QSPLIT = _repack_split(V)
    nb = QSPLIT // BN
    last = V // BN

    def body(x0_ref, x1_ref, x2_ref, x3_ref, o_ref):
        xs = [x0_ref[...], x1_ref[...], x2_ref[...], x3_ref[...]]
        half = EMB // 2
        m = jnp.concatenate(
            [x[:half] for x in xs] + [x[half:] for x in xs], axis=0)
        z = jnp.transpose(m)
        u = lax.bitcast_convert_type(z, jnp.uint32)
        r = (u + jnp.uint32(0x7FFF) + ((u >> 16) & jnp.uint32(1))) >> 16
        o_ref[...] = r[:, :2 * EMB] | (r[:, 2 * EMB:] << 16)

    def make_map(q):
        if q == 0:
            return lambda i: (0, i)
        return lambda i, q=q: (0, jnp.minimum(i + q * nb, last))

    return pl.pallas_call(
        body,
        grid=(nb,),
        in_specs=[pl.BlockSpec((EMB, BN), make_map(q)) for q in range(_NQ)],
        out_specs=pl.BlockSpec((BN, 2 * EMB), lambda i: (i, 0)),
        out_shape=jax.ShapeDtypeStruct((QSPLIT, 2 * EMB), jnp.uint32),
    )(tableT, tableT, tableT, tableT)


def _mlp(x, w1t, b1, w2t, b2):
    B, EMB = x.shape
    HID = w1t.shape[1]
    OUT = w2t.shape[1]
    BM = 2048

    def body(x_ref, w1_ref, b1_ref, w2_ref, b2_ref, o_ref):
        h = jnp.dot(x_ref[...], w1_ref[...],
                    preferred_element_type=jnp.float32)
        h = jnp.maximum(h + b1_ref[...], 0.0)
        o = jnp.dot(h, w2_ref[...], preferred_element_type=jnp.float32)
        o_ref[...] = jnp.maximum(o + b2_ref[...], 0.0)

    return pl.pallas_call(
        body,
        grid=(B // BM,),
        in_specs=[
            pl.BlockSpec((BM, EMB), lambda i: (i, 0)),
            pl.BlockSpec((EMB, HID), lambda i: (0, 0)),
            pl.BlockSpec((1, HID), lambda i: (0, 0)),
            pl.BlockSpec((HID, OUT), lambda i: (0, 0)),
            pl.BlockSpec((1, OUT), lambda i: (0, 0)),
        ],
        out_specs=pl.BlockSpec((BM, OUT), lambda i: (i, 0)),
        out_shape=jax.ShapeDtypeStruct((B, OUT), jnp.float32),
    )(x, w1t, b1.reshape(1, -1), w2t, b2.reshape(1, -1))


_COL_PERM = np.array(
    [*range(0, 16), *range(32, 48),
     *range(16, 32), *range(48, 64)], dtype=np.int32)


def kernel(inputs, batch_size, emb_table, W1, b1, W2, b2):
    B, L = inputs.shape
    V, EMB = emb_table.shape
    QS = _repack_split(V)
    idx2d = _prep_idx(inputs.T, QS)
    table_lin = _repack_table(emb_table.T).reshape(_NQ * QS, EMB // 2)
    embeds_p = _sc_embed_sum(idx2d, table_lin, B, L)
    w1t_p = W1.T[jnp.asarray(_COL_PERM)]
    return _mlp(embeds_p, w1t_p, b1, W2.T, b2)


# speedup vs baseline: 10.2988x; 10.2988x over previous
import functools

import jax
import jax.numpy as jnp
import numpy as np
from jax import lax
from jax.experimental import pallas as pl
from jax.experimental.pallas import tpu as pltpu
from jax.experimental.pallas import tpu_sc as plsc

NC = 2
NS = 16
NW = NC * NS
LANES = 16


@functools.partial(jax.jit, static_argnums=(2, 3))
def _sc_embed_sum(idx2d, table, B, L):
    W = table.shape[1]
    EMB = 2 * W
    EPW = B // NW
    CH_E = 16
    CH_R = CH_E * L
    NCH = EPW // CH_E
    subs = []
    off = 0
    while off < CH_R:
        sz = min(128, CH_R - off)
        subs.append((off, sz))
        off += sz

    mesh = plsc.VectorSubcoreMesh(core_axis_name="c", subcore_axis_name="s")

    @functools.partial(
        pl.kernel,
        out_type=jax.ShapeDtypeStruct((B, EMB), jnp.float32),
        mesh=mesh,
        scratch_types=[
            pltpu.VMEM((4, CH_R), jnp.int32),
            pltpu.VMEM((4, CH_R, W), jnp.uint32),
            pltpu.VMEM((4, CH_E, EMB), jnp.float32),
        ] + [pltpu.SemaphoreType.DMA] * 12,
        compiler_params=pltpu.CompilerParams(
            use_tc_tiling_on_sc=False, needs_layout_passes=False),
    )
    def sc_kernel(idx_hbm, tab_hbm, out_hbm, idxs, rows, outs, *sems):
        wid = lax.axis_index("s") * NC + lax.axis_index("c")
        row_base = wid * NCH
        out_base = wid * EPW
        si = sems[0:4]
        sg = sems[4:8]
        so = sems[8:12]
        himask = jnp.uint32(0xFFFF0000)

        def issue_idx(c, s):
            pltpu.async_copy(idx_hbm.at[row_base + c], idxs.at[s], si[s])

        def wait_idx(s):
            pltpu.make_async_copy(idx_hbm.at[0], idxs.at[s], si[s]).wait()

        def issue_gathers(s):
            for (o, sz) in subs:
                pltpu.async_copy(
                    tab_hbm.at[idxs.at[s].at[pl.ds(o, sz)]],
                    rows.at[s].at[pl.ds(o, sz)],
                    sg[s])

        def wait_gathers(s):
            pltpu.make_async_copy(
                tab_hbm.at[pl.ds(0, CH_R)], rows.at[s], sg[s]).wait()

        def issue_out(c, s):
            pltpu.async_copy(
                outs.at[s], out_hbm.at[pl.ds(out_base + c * CH_E, CH_E)],
                so[s])

        def wait_out(s):
            pltpu.make_async_copy(
                outs.at[s], out_hbm.at[pl.ds(0, CH_E)], so[s]).wait()

        def expand(w):
            lo = plsc.bitcast(w << 16, jnp.float32)
            hi = plsc.bitcast(w & himask, jnp.float32)
            return lo, hi

        def reduce_chunk(s):
            rows_s = rows.at[s]
            outs_s = outs.at[s]

            @pl.loop(0, CH_E)
            def _(e):
                r0 = e * L
                accs = [None] * (2 * W // LANES)
                accs2 = [None] * (2 * W // LANES)
                for j in range(L):
                    tgt = accs if (j & 1) == 0 else accs2
                    for h in range(W // LANES):
                        lo, hi = expand(
                            rows_s[r0 + j, pl.ds(h * LANES, LANES)])
                        if tgt[2 * h] is None:
                            tgt[2 * h] = lo
                            tgt[2 * h + 1] = hi
                        else:
                            tgt[2 * h] = tgt[2 * h] + lo
                            tgt[2 * h + 1] = tgt[2 * h + 1] + hi
                for k in range(len(accs)):
                    outs_s[e, pl.ds(k * LANES, LANES)] = accs[k] + accs2[k]

        for k in range(4):
            issue_idx(k, k)
        wait_idx(0)
        issue_gathers(0)
        wait_idx(1)
        issue_gathers(1)

        @pl.loop(0, NCH, step=4)
        def _(cbase):
            for b in range(4):
                c = cbase + b
                s = b
                wait_gathers(s)

                @pl.when(c + 4 < NCH)
                def _():
                    issue_idx(c + 4, s)

                @pl.when(c + 2 < NCH)
                def _():
                    wait_idx((b + 2) % 4)
                    issue_gathers((b + 2) % 4)

                @pl.when(c >= 4)
                def _():
                    wait_out(s)

                reduce_chunk(s)
                issue_out(c, s)

        for k in range(4):
            wait_out(k)

    return sc_kernel(idx2d, table)


_REPACK_BN = 15872
_NQ = 4


def _repack_split(V):
    nb = -(-V // (_NQ * _REPACK_BN))
    return nb * _REPACK_BN


def _repack_table(tableT):
    EMB, V = tableT.shape
    BN = _REPACK_BN
    QSPLIT = _repack_split(V)
    nb = QSPLIT // BN
    last = V // BN

    def body(x0_ref, x1_ref, x2_ref, x3_ref, o_ref):
        xs = [x0_ref[...], x1_ref[...], x2_ref[...], x3_ref[...]]
        half = EMB // 2
        m = jnp.concatenate(
            [x[:half] for x in xs] + [x[half:] for x in xs], axis=0)
        z = jnp.transpose(m)
        u = lax.bitcast_convert_type(z, jnp.uint32)
        r = (u + jnp.uint32(0x7FFF) + ((u >> 16) & jnp.uint32(1))) >> 16
        o_ref[...] = r[:, :2 * EMB] | (r[:, 2 * EMB:] << 16)

    def make_map(q):
        if q == 0:
            return lambda i: (0, i)
        return lambda i, q=q: (0, jnp.minimum(i + q * nb, last))

    return pl.pallas_call(
        body,
        grid=(nb,),
        in_specs=[pl.BlockSpec((EMB, BN), make_map(q)) for q in range(_NQ)],
        out_specs=pl.BlockSpec((BN, 2 * EMB), lambda i: (i, 0)),
        out_shape=jax.ShapeDtypeStruct((QSPLIT, 2 * EMB), jnp.uint32),
    )(tableT, tableT, tableT, tableT)


def _mlp(x, w1t, b1, w2t, b2):
    B, EMB = x.shape
    HID = w1t.shape[1]
    OUT = w2t.shape[1]
    BM = 2048

    def body(x_ref, w1_ref, b1_ref, w2_ref, b2_ref, o_ref):
        h = jnp.dot(x_ref[...], w1_ref[...],
                    preferred_element_type=jnp.float32)
        h = jnp.maximum(h + b1_ref[...], 0.0)
        o = jnp.dot(h, w2_ref[...], preferred_element_type=jnp.float32)
        o_ref[...] = jnp.maximum(o + b2_ref[...], 0.0)

    return pl.pallas_call(
        body,
        grid=(B // BM,),
        in_specs=[
            pl.BlockSpec((BM, EMB), lambda i: (i, 0)),
            pl.BlockSpec((EMB, HID), lambda i: (0, 0)),
            pl.BlockSpec((1, HID), lambda i: (0, 0)),
            pl.BlockSpec((HID, OUT), lambda i: (0, 0)),
            pl.BlockSpec((1, OUT), lambda i: (0, 0)),
        ],
        out_specs=pl.BlockSpec((BM, OUT), lambda i: (i, 0)),
        out_shape=jax.ShapeDtypeStruct((B, OUT), jnp.float32),
    )(x, w1t, b1.reshape(1, -1), w2t, b2.reshape(1, -1))


_COL_PERM = np.array(
    [*range(0, 16), *range(32, 48),
     *range(16, 32), *range(48, 64)], dtype=np.int32)


def kernel(inputs, batch_size, emb_table, W1, b1, W2, b2):
    B, L = inputs.shape
    V, EMB = emb_table.shape
    QS = _repack_split(V)
    q = inputs // QS
    idx_r = 4 * (inputs - q * QS) + q
    idx2d = idx_r.reshape(-1, 16 * L)
    table_lin = _repack_table(emb_table.T).reshape(_NQ * QS, EMB // 2)
    embeds_p = _sc_embed_sum(idx2d, table_lin, B, L)
    w1t_p = W1.T[jnp.asarray(_COL_PERM)]
    return _mlp(embeds_p, w1t_p, b1, W2.T, b2)
